# baseline (speedup 1.0000x reference)
import jax
import jax.numpy as jnp
from jax import lax
from jax.experimental import pallas as pl
from jax.experimental.pallas import tpu as pltpu
from jax.experimental.pallas import tpu_sc as plsc

BOHR_TO_ANGSTROM = 0.529177210544
ANGSTROM_TO_BOHR = 1.0 / BOHR_TO_ANGSTROM
HARTREE_TO_EV = 27.211386245981
A1 = 0.3981
A2 = 4.4211
S8 = 1.9889
S6 = 1.0
K1 = 16.0
K3 = -4.0


def _rsqrt(x):
    i = plsc.bitcast(x, jnp.int32)
    i = jnp.int32(0x5F3759DF) - lax.shift_right_arithmetic(i, 1)
    y = plsc.bitcast(i, jnp.float32)
    for _ in range(2):
        y = y * (1.5 - 0.5 * x * y * y)
    return y


def _splat(ref, i):
    return plsc.load_gather(ref, [jnp.full((16,), i, jnp.int32)])


def kernel(positions, numbers, neighbor_matrix, covalent_radii, r4r2,
           c6_reference, coord_num_ref):
    N, NBRS = neighbor_matrix.shape
    NZ = covalent_radii.shape[0]
    NRF = coord_num_ref.shape[1]
    KV = NBRS // 16
    try:
        info = plsc.get_sparse_core_info()
        NC, NS = info.num_cores, info.num_subcores
    except ValueError:
        NC, NS = 2, 16
    NW = NC * NS
    GROUP = 64
    N_PAD = -(-N // (NW * GROUP)) * (NW * GROUP)
    APW = N_PAD // NW
    NG = APW // GROUP
    GP = GROUP * NBRS
    NZR = -(-NZ // 32) * 32
    C6W = -(-(NRF * NRF * NZR) // 128) * 128
    NZP = 128
    CNW = NRF * NZR
    FSCALE = -(HARTREE_TO_EV / BOHR_TO_ANGSTROM)

    mesh = plsc.VectorSubcoreMesh(core_axis_name="c", subcore_axis_name="s",
                                  num_cores=NC, num_subcores=NS)
    cparams = pltpu.CompilerParams(needs_layout_passes=False)
    f32, i32 = jnp.float32, jnp.int32

    posb = positions.astype(f32) * ANGSTROM_TO_BOHR
    pad = N_PAD - N
    posx = jnp.pad(posb[:, 0], (0, pad))
    posy = jnp.pad(posb[:, 1], (0, pad))
    posz = jnp.pad(posb[:, 2], (0, pad))
    num = jnp.pad(numbers.astype(i32), (0, pad))
    nbrf = jnp.pad(neighbor_matrix.astype(i32), ((0, pad), (0, 0)),
                   constant_values=N).reshape(-1)
    cv = jnp.pad(covalent_radii.astype(f32), (0, NZP - NZ))
    r4 = jnp.pad(r4r2.astype(f32), (0, NZP - NZ))
    cnr = jnp.pad(coord_num_ref.astype(f32).T,
                  ((0, 0), (0, NZR - NZ))).reshape(-1)
    c6r = jnp.pad(
        jnp.pad(jnp.transpose(
            c6_reference.astype(f32).reshape(NZ, NZ, NRF * NRF), (0, 2, 1)),
            ((0, 0), (0, 0), (0, NZR - NZ))).reshape(NZ, NRF * NRF * NZR),
        ((0, 0), (0, C6W - NRF * NRF * NZR)))

    def wid_of():
        return lax.axis_index("s") * NC + lax.axis_index("c")

    def k1_body(posx_h, posy_h, posz_h, num_h, nbr_h, cv_h,
                cn_out, invr_h, s2_h, ux_h, uy_h, uz_h,
                px, py, pz, nm, nb, cvv, stage, invrf, s2f, uxf, uyf, uzf):
        wid = wid_of()
        base = wid * APW
        pltpu.sync_copy(posx_h, px)
        pltpu.sync_copy(posy_h, py)
        pltpu.sync_copy(posz_h, pz)
        pltpu.sync_copy(num_h, nm)
        pltpu.sync_copy(cv_h, cvv)
        pltpu.sync_copy(nbr_h.at[pl.ds(base * NBRS, APW * NBRS)], nb)
        lane = lax.iota(i32, 16)
        m0 = lane == 0

        def atom(a):
            i = base + a
            t = lax.rem(a, GROUP)
            xi = _splat(px, i)
            yi = _splat(py, i)
            zi = _splat(pz, i)
            zn = _splat(nm, i)
            rci = plsc.load_gather(cvv, [zn])
            acc = jnp.zeros((16,), f32)
            for k in range(KV):
                j = nb[pl.ds(a * NBRS + k * 16, 16)]
                valid = (j < N) & (j != i)
                jc = jnp.where(valid, j, 0)
                xj = plsc.load_gather(px, [jc])
                yj = plsc.load_gather(py, [jc])
                zj = plsc.load_gather(pz, [jc])
                znj = plsc.load_gather(nm, [jc])
                rcj = plsc.load_gather(cvv, [znj])
                dx = xj - xi
                dy = yj - yi
                dz = zj - zi
                d2 = dx * dx + dy * dy + dz * dz
                d2s = jnp.where(valid, d2, 1.0)
                invr = _rsqrt(d2s)
                rc = rci + rcj
                ex = jnp.exp(-K1 * (rc * invr - 1.0))
                sg = 1.0 / (1.0 + ex)
                acc = acc + jnp.where(valid, sg, 0.0)
                off = t * NBRS + k * 16
                invrf[pl.ds(off, 16)] = invr
                s2f[pl.ds(off, 16)] = jnp.where(
                    valid, -K1 * rc * invr * invr * sg * (1.0 - sg), 0.0)
                uxf[pl.ds(off, 16)] = dx * invr
                uyf[pl.ds(off, 16)] = dy * invr
                uzf[pl.ds(off, 16)] = dz * invr
            cni = jnp.sum(acc)
            plsc.store_scatter(stage, [jnp.full((16,), a, i32)],
                               jnp.full((16,), cni, f32), mask=m0)

        def group(g, _):
            plsc.parallel_loop(g * GROUP, (g + 1) * GROUP)(atom)
            dst = (base + g * GROUP) * NBRS
            pltpu.sync_copy(invrf, invr_h.at[pl.ds(dst, GP)])
            pltpu.sync_copy(s2f, s2_h.at[pl.ds(dst, GP)])
            pltpu.sync_copy(uxf, ux_h.at[pl.ds(dst, GP)])
            pltpu.sync_copy(uyf, uy_h.at[pl.ds(dst, GP)])
            pltpu.sync_copy(uzf, uz_h.at[pl.ds(dst, GP)])
            return 0

        lax.fori_loop(0, NG, group, 0)
        pltpu.sync_copy(stage, cn_out.at[pl.ds(base, APW)])

    pair_sds = jax.ShapeDtypeStruct((N_PAD * NBRS,), f32)
    k1 = pl.kernel(
        k1_body,
        out_type=(jax.ShapeDtypeStruct((N_PAD,), f32),
                  pair_sds, pair_sds, pair_sds, pair_sds, pair_sds),
        mesh=mesh,
        scratch_types=[
            pltpu.VMEM((N_PAD,), f32), pltpu.VMEM((N_PAD,), f32),
            pltpu.VMEM((N_PAD,), f32), pltpu.VMEM((N_PAD,), i32),
            pltpu.VMEM((APW * NBRS,), i32), pltpu.VMEM((NZP,), f32),
            pltpu.VMEM((APW,), f32),
            pltpu.VMEM((GP,), f32), pltpu.VMEM((GP,), f32),
            pltpu.VMEM((GP,), f32), pltpu.VMEM((GP,), f32),
            pltpu.VMEM((GP,), f32),
        ],
        compiler_params=cparams,
    )
    cn_flat, invr_p, s2_p, ux_p, uy_p, uz_p = k1(posx, posy, posz, num,
                                                 nbrf, cv)

    def k2_body(num_h, nbr_h, r4_h, cnr_h, c6r_h, cn_h, invr_h,
                gpart, epart, s1_h,
                nm, nb, r4v, cnv, cnrv, c6buf, gpriv, s1f, invrf, ebuf, sem):
        wid = wid_of()
        base = wid * APW
        pltpu.sync_copy(num_h, nm)
        pltpu.sync_copy(r4_h, r4v)
        pltpu.sync_copy(cnr_h, cnrv)
        pltpu.sync_copy(cn_h, cnv)
        pltpu.sync_copy(nbr_h.at[pl.ds(base * NBRS, APW * NBRS)], nb)
        lane = lax.iota(i32, 16)
        m0 = lane == 0
        lm5 = lane < NRF
        a96 = jnp.where(lm5, lane, NRF - 1) * NZR

        def zero(r, _):
            gpriv[pl.ds(r * 16, 16)] = jnp.zeros((16,), f32)
            return 0
        lax.fori_loop(0, N_PAD // 16, zero, 0)

        def slab(slot):
            return c6buf.at[pl.ds(slot * C6W, C6W)]

        CH = 8
        NCHT = APW // CH

        def issue(q):
            for t in range(CH):
                zi0 = _splat(nm, base + q * CH + t)[0]
                pltpu.make_async_copy(
                    c6r_h.at[zi0], slab(lax.rem(q * CH + t, 2 * CH)),
                    sem).start()

        issue(0)

        def atom(a, eacc):
            i = base + a
            slot = lax.rem(a, 2 * CH)
            t_in_g = lax.rem(a, GROUP)
            zi_v = _splat(nm, i)
            cni = _splat(cnv, i)
            r4i = plsc.load_gather(r4v, [zi_v])
            cnref_i = plsc.load_gather(cnrv, [zi_v + a96])
            dci = cni - cnref_i
            vLi = jnp.exp(K3 * dci * dci)
            vdLi = (2.0 * K3) * dci * vLi
            wi = jnp.sum(jnp.where(lm5, vLi, 0.0))
            dwi = jnp.sum(jnp.where(lm5, vdLi, 0.0))
            Li = [vLi[t] for t in range(NRF)]
            dLi = [vdLi[t] for t in range(NRF)]
            gacc = jnp.zeros((16,), f32)
            for k in range(KV):
                j = nb[pl.ds(a * NBRS + k * 16, 16)]
                valid = (j < N) & (j != i)
                jc = jnp.where(valid, j, 0)
                znj = plsc.load_gather(nm, [jc])
                cnj = plsc.load_gather(cnv, [jc])
                r4j = plsc.load_gather(r4v, [znj])
                invr = invrf[pl.ds(t_in_g * NBRS + k * 16, 16)]
                Lj = []
                dLj = []
                wj = jnp.zeros((16,), f32)
                dwj = jnp.zeros((16,), f32)
                for b in range(NRF):
                    cnref_j = plsc.load_gather(
                        cnrv.at[pl.ds(b * NZR, NZR)], [znj])
                    dcj = cnj - cnref_j
                    ljb = jnp.exp(K3 * dcj * dcj)
                    dljb = (2.0 * K3) * dcj * ljb
                    wj = wj + ljb
                    dwj = dwj + dljb
                    Lj.append(ljb)
                    dLj.append(dljb)
                cslab = slot * C6W
                nsum = jnp.zeros((16,), f32)
                dni = jnp.zeros((16,), f32)
                dnj = jnp.zeros((16,), f32)
                for t in range(NRF):
                    inner = jnp.zeros((16,), f32)
                    innerd = jnp.zeros((16,), f32)
                    for b in range(NRF):
                        ab = t * NRF + b
                        C = plsc.load_gather(
                            c6buf.at[pl.ds(cslab + ab * NZR, NZR)], [znj])
                        inner = inner + C * Lj[b]
                        innerd = innerd + C * dLj[b]
                    nsum = nsum + Li[t] * inner
                    dni = dni + dLi[t] * inner
                    dnj = dnj + Li[t] * innerd
                W = wi * wj
                Wc = jnp.maximum(W, 1e-30)
                live = W > 1e-30
                qq = 3.0 * r4i * r4j
                r0 = qq * _rsqrt(qq)
                fd = A1 * r0 + A2
                f2 = fd * fd
                f6 = f2 * f2 * f2
                f8 = f6 * f2
                i2 = invr * invr
                i6 = i2 * i2 * i2
                i8 = i6 * i2
                dA = 1.0 + f6 * i6
                dB = 1.0 + f8 * i8
                iP6 = 1.0 / (dA * Wc)
                invWc = dA * iP6
                u6 = Wc * iP6
                u8 = 1.0 / dB
                c6 = nsum * invWc
                dc6i = jnp.where(live, (dni - c6 * dwi * wj) * invWc, 0.0)
                dc6j = jnp.where(live, (dnj - c6 * wi * dwj) * invWc, 0.0)
                e6 = c6 * i6 * u6
                e8 = qq * c6 * i8 * u8
                eacc = eacc - jnp.where(valid, S6 * e6 + S8 * e8, 0.0)
                g6 = jnp.where(valid,
                               -0.5 * (S6 * i6 * u6 + S8 * qq * i8 * u8), 0.0)
                gacc = gacc + g6 * dc6i
                plsc.addupdate_scatter(gpriv, [jc], g6 * dc6j)
                s1v = jnp.where(
                    valid,
                    0.5 * (6.0 * S6 * e6 * u6 + 8.0 * S8 * e8 * u8) * invr,
                    0.0)
                s1f[pl.ds(t_in_g * NBRS + k * 16, 16)] = s1v
            gi = jnp.sum(gacc)
            plsc.addupdate_scatter(gpriv, [jnp.full((16,), i, i32)],
                                   jnp.full((16,), gi, f32), mask=m0)
            return eacc

        def chunk(q, eacc):
            for _ in range(CH):
                pltpu.make_async_copy(c6r_h.at[0], slab(0), sem).wait()
            eacc = plsc.parallel_loop(q * CH, (q + 1) * CH,
                                      carry=eacc)(atom)
            issue(jnp.minimum(q + 1, NCHT - 1))
            return eacc

        def group(g, eacc):
            src = (base + g * GROUP) * NBRS
            pltpu.sync_copy(invr_h.at[pl.ds(src, GP)], invrf)
            eacc = lax.fori_loop(g * (GROUP // CH), (g + 1) * (GROUP // CH),
                                 chunk, eacc)
            pltpu.sync_copy(s1f, s1_h.at[pl.ds(src, GP)])
            return eacc

        eacc = lax.fori_loop(0, NG, group, jnp.zeros((16,), f32))
        for _ in range(CH):
            pltpu.make_async_copy(c6r_h.at[0], slab(0), sem).wait()
        ebuf[...] = eacc
        pltpu.sync_copy(ebuf, epart.at[pl.ds(wid * 16, 16)])
        pltpu.sync_copy(gpriv, gpart.at[pl.ds(wid * N_PAD, N_PAD)])

    k2 = pl.kernel(
        k2_body,
        out_type=(
            jax.ShapeDtypeStruct((NW * N_PAD,), f32),
            jax.ShapeDtypeStruct((NW * 16,), f32),
            pair_sds,
        ),
        mesh=mesh,
        scratch_types=[
            pltpu.VMEM((N_PAD,), i32), pltpu.VMEM((APW * NBRS,), i32),
            pltpu.VMEM((NZP,), f32), pltpu.VMEM((N_PAD,), f32),
            pltpu.VMEM((CNW,), f32), pltpu.VMEM((16 * C6W,), f32),
            pltpu.VMEM((N_PAD,), f32),
            pltpu.VMEM((GP,), f32), pltpu.VMEM((GP,), f32),
            pltpu.VMEM((16,), f32),
            pltpu.SemaphoreType.DMA,
        ],
        compiler_params=cparams,
    )
    gpart, epart, s1_p = k2(num, nbrf, r4, cnr, c6r, cn_flat, invr_p)

    def k3_body(nbr_h, s1_h, s2_h, ux_h, uy_h, uz_h, gpart_h,
                fpart,
                nb, gall, gown, gx, gy, gz, s1f, s2f, uxf, uyf, uzf, sem):
        wid = wid_of()
        base = wid * APW
        pltpu.sync_copy(nbr_h.at[pl.ds(base * NBRS, APW * NBRS)], nb)
        lane = lax.iota(i32, 16)
        m0 = lane == 0
        for s in range(NW):
            pltpu.make_async_copy(
                gpart_h.at[pl.ds(s * N_PAD + base, APW)],
                gall.at[pl.ds(s * APW, APW)], sem).start()
        for s in range(NW):
            pltpu.make_async_copy(
                gpart_h.at[pl.ds(base, APW)],
                gall.at[pl.ds(s * APW, APW)], sem).wait()

        def red(r, _):
            acc = jnp.zeros((16,), f32)
            for s in range(NW):
                acc = acc + gall[pl.ds(s * APW + r * 16, 16)]
            gown[pl.ds(r * 16, 16)] = acc
            return 0
        lax.fori_loop(0, APW // 16, red, 0)

        def zero(r, _):
            z = jnp.zeros((16,), f32)
            gx[pl.ds(r * 16, 16)] = z
            gy[pl.ds(r * 16, 16)] = z
            gz[pl.ds(r * 16, 16)] = z
            return 0
        lax.fori_loop(0, N_PAD // 16, zero, 0)

        def atom(a):
            i = base + a
            t_in_g = lax.rem(a, GROUP)
            Gi = _splat(gown, a)
            accx = jnp.zeros((16,), f32)
            accy = jnp.zeros((16,), f32)
            accz = jnp.zeros((16,), f32)
            for k in range(KV):
                off = t_in_g * NBRS + k * 16
                j = nb[pl.ds(a * NBRS + k * 16, 16)]
                F = s1f[pl.ds(off, 16)] + Gi * s2f[pl.ds(off, 16)]
                cx = F * uxf[pl.ds(off, 16)]
                cy = F * uyf[pl.ds(off, 16)]
                cz = F * uzf[pl.ds(off, 16)]
                accx = accx + cx
                accy = accy + cy
                accz = accz + cz
                plsc.addupdate_scatter(gx, [j], cx)
                plsc.addupdate_scatter(gy, [j], cy)
                plsc.addupdate_scatter(gz, [j], cz)
            iv = jnp.full((16,), i, i32)
            plsc.addupdate_scatter(gx, [iv],
                                   jnp.full((16,), -jnp.sum(accx), f32),
                                   mask=m0)
            plsc.addupdate_scatter(gy, [iv],
                                   jnp.full((16,), -jnp.sum(accy), f32),
                                   mask=m0)
            plsc.addupdate_scatter(gz, [iv],
                                   jnp.full((16,), -jnp.sum(accz), f32),
                                   mask=m0)

        def group(g, _):
            src = (base + g * GROUP) * NBRS
            pltpu.sync_copy(s1_h.at[pl.ds(src, GP)], s1f)
            pltpu.sync_copy(s2_h.at[pl.ds(src, GP)], s2f)
            pltpu.sync_copy(ux_h.at[pl.ds(src, GP)], uxf)
            pltpu.sync_copy(uy_h.at[pl.ds(src, GP)], uyf)
            pltpu.sync_copy(uz_h.at[pl.ds(src, GP)], uzf)
            plsc.parallel_loop(g * GROUP, (g + 1) * GROUP)(atom)
            return 0

        lax.fori_loop(0, NG, group, 0)
        pltpu.sync_copy(gx, fpart.at[pl.ds(wid * 3 * N_PAD, N_PAD)])
        pltpu.sync_copy(gy, fpart.at[pl.ds(wid * 3 * N_PAD + N_PAD, N_PAD)])
        pltpu.sync_copy(gz, fpart.at[pl.ds(wid * 3 * N_PAD + 2 * N_PAD, N_PAD)])

    k3 = pl.kernel(
        k3_body,
        out_type=jax.ShapeDtypeStruct((NW * 3 * N_PAD,), f32),
        mesh=mesh,
        scratch_types=[
            pltpu.VMEM((APW * NBRS,), i32),
            pltpu.VMEM((NW * APW,), f32), pltpu.VMEM((APW,), f32),
            pltpu.VMEM((N_PAD,), f32), pltpu.VMEM((N_PAD,), f32),
            pltpu.VMEM((N_PAD,), f32),
            pltpu.VMEM((GP,), f32), pltpu.VMEM((GP,), f32),
            pltpu.VMEM((GP,), f32), pltpu.VMEM((GP,), f32),
            pltpu.VMEM((GP,), f32),
            pltpu.SemaphoreType.DMA,
        ],
        compiler_params=cparams,
    )
    fpart = k3(nbrf, s1_p, s2_p, ux_p, uy_p, uz_p, gpart)

    def k4_body(fpart_h, epart_h, f3, e16, fall, facc, evbuf, ebuf, sem):
        wid = wid_of()
        base = wid * APW
        for c in range(3):
            for s in range(NW):
                pltpu.make_async_copy(
                    fpart_h.at[pl.ds(s * 3 * N_PAD + c * N_PAD + base, APW)],
                    fall.at[pl.ds(s * APW, APW)], sem).start()
            for s in range(NW):
                pltpu.make_async_copy(
                    fpart_h.at[pl.ds(base, APW)],
                    fall.at[pl.ds(s * APW, APW)], sem).wait()

            def red(r, _):
                acc = jnp.zeros((16,), f32)
                for s in range(NW):
                    acc = acc + fall[pl.ds(s * APW + r * 16, 16)]
                facc[pl.ds(r * 16, 16)] = acc * FSCALE
                return 0
            lax.fori_loop(0, APW // 16, red, 0)
            pltpu.sync_copy(facc, f3.at[pl.ds(c * N_PAD + base, APW)])

        @pl.when(wid == 0)
        def _():
            pltpu.sync_copy(epart_h, evbuf)
            acc = jnp.zeros((16,), f32)
            for s in range(NW):
                acc = acc + evbuf[pl.ds(s * 16, 16)]
            e = 0.5 * HARTREE_TO_EV * jnp.sum(acc)
            ebuf[...] = jnp.full((16,), e, f32)
            pltpu.sync_copy(ebuf, e16)

    k4 = pl.kernel(
        k4_body,
        out_type=(
            jax.ShapeDtypeStruct((3 * N_PAD,), f32),
            jax.ShapeDtypeStruct((16,), f32),
        ),
        mesh=mesh,
        scratch_types=[
            pltpu.VMEM((NW * APW,), f32), pltpu.VMEM((APW,), f32),
            pltpu.VMEM((NW * 16,), f32), pltpu.VMEM((16,), f32),
            pltpu.SemaphoreType.DMA,
        ],
        compiler_params=cparams,
    )
    f3, e16 = k4(fpart, epart)

    forces = jnp.transpose(f3.reshape(3, N_PAD))[:N]
    energy = e16[:1]
    cn = cn_flat[:N]
    return (energy, forces, cn)


# speedup vs baseline: 1.1209x; 1.1209x over previous
import jax
import jax.numpy as jnp
from jax import lax
from jax.experimental import pallas as pl
from jax.experimental.pallas import tpu as pltpu
from jax.experimental.pallas import tpu_sc as plsc

BOHR_TO_ANGSTROM = 0.529177210544
ANGSTROM_TO_BOHR = 1.0 / BOHR_TO_ANGSTROM
HARTREE_TO_EV = 27.211386245981
A1 = 0.3981
A2 = 4.4211
S8 = 1.9889
S6 = 1.0
K1 = 16.0
K3 = -4.0


def _rsqrt(x):
    i = plsc.bitcast(x, jnp.int32)
    i = jnp.int32(0x5F3759DF) - lax.shift_right_arithmetic(i, 1)
    y = plsc.bitcast(i, jnp.float32)
    for _ in range(2):
        y = y * (1.5 - 0.5 * x * y * y)
    return y


def _splat(ref, i):
    return plsc.load_gather(ref, [jnp.full((16,), i, jnp.int32)])


def kernel(positions, numbers, neighbor_matrix, covalent_radii, r4r2,
           c6_reference, coord_num_ref):
    N, NBRS = neighbor_matrix.shape
    NZ = covalent_radii.shape[0]
    NRF = coord_num_ref.shape[1]
    KV = NBRS // 16
    try:
        info = plsc.get_sparse_core_info()
        NC, NS = info.num_cores, info.num_subcores
    except ValueError:
        NC, NS = 2, 16
    NW = NC * NS
    GROUP = 64
    N_PAD = -(-N // (NW * GROUP)) * (NW * GROUP)
    APW = N_PAD // NW
    NG = APW // GROUP
    GP = GROUP * NBRS
    NZR = -(-NZ // 32) * 32
    C6W = -(-(NRF * NRF * NZR) // 128) * 128
    NZP = 128
    CNW = NRF * NZR
    FSCALE = -(HARTREE_TO_EV / BOHR_TO_ANGSTROM)

    mesh = plsc.VectorSubcoreMesh(core_axis_name="c", subcore_axis_name="s",
                                  num_cores=NC, num_subcores=NS)
    cparams = pltpu.CompilerParams(needs_layout_passes=False)
    f32, i32 = jnp.float32, jnp.int32

    posb = positions.astype(f32) * ANGSTROM_TO_BOHR
    pad = N_PAD - N
    posx = jnp.pad(posb[:, 0], (0, pad))
    posy = jnp.pad(posb[:, 1], (0, pad))
    posz = jnp.pad(posb[:, 2], (0, pad))
    num = jnp.pad(numbers.astype(i32), (0, pad))
    nbrf = jnp.pad(neighbor_matrix.astype(i32), ((0, pad), (0, 0)),
                   constant_values=N).reshape(-1)
    cv = jnp.pad(covalent_radii.astype(f32), (0, NZP - NZ))
    r4 = jnp.pad(r4r2.astype(f32), (0, NZP - NZ))
    cnr = jnp.pad(coord_num_ref.astype(f32).T,
                  ((0, 0), (0, NZR - NZ))).reshape(-1)
    c6r = jnp.pad(
        jnp.pad(jnp.transpose(
            c6_reference.astype(f32).reshape(NZ, NZ, NRF * NRF), (0, 2, 1)),
            ((0, 0), (0, 0), (0, NZR - NZ))).reshape(NZ, NRF * NRF * NZR),
        ((0, 0), (0, C6W - NRF * NRF * NZR)))

    def wid_of():
        return lax.axis_index("s") * NC + lax.axis_index("c")

    def k1_body(posx_h, posy_h, posz_h, num_h, nbr_h, cv_h,
                cn_out, invr_h, s2_h, ux_h, uy_h, uz_h,
                px, py, pz, nm, nb, cvv, stage, invrf, s2f, uxf, uyf, uzf):
        wid = wid_of()
        base = wid * APW
        pltpu.sync_copy(posx_h, px)
        pltpu.sync_copy(posy_h, py)
        pltpu.sync_copy(posz_h, pz)
        pltpu.sync_copy(num_h, nm)
        pltpu.sync_copy(cv_h, cvv)
        pltpu.sync_copy(nbr_h.at[pl.ds(base * NBRS, APW * NBRS)], nb)
        lane = lax.iota(i32, 16)
        m0 = lane == 0

        def atom(a):
            i = base + a
            t = lax.rem(a, GROUP)
            xi = _splat(px, i)
            yi = _splat(py, i)
            zi = _splat(pz, i)
            zn = _splat(nm, i)
            rci = plsc.load_gather(cvv, [zn])
            acc = jnp.zeros((16,), f32)
            for k in range(KV):
                j = nb[pl.ds(a * NBRS + k * 16, 16)]
                valid = (j < N) & (j != i)
                jc = jnp.where(valid, j, 0)
                xj = plsc.load_gather(px, [jc])
                yj = plsc.load_gather(py, [jc])
                zj = plsc.load_gather(pz, [jc])
                znj = plsc.load_gather(nm, [jc])
                rcj = plsc.load_gather(cvv, [znj])
                dx = xj - xi
                dy = yj - yi
                dz = zj - zi
                d2 = dx * dx + dy * dy + dz * dz
                d2s = jnp.where(valid, d2, 1.0)
                invr = _rsqrt(d2s)
                rc = rci + rcj
                ex = jnp.exp(-K1 * (rc * invr - 1.0))
                sg = 1.0 / (1.0 + ex)
                acc = acc + jnp.where(valid, sg, 0.0)
                off = t * NBRS + k * 16
                invrf[pl.ds(off, 16)] = invr
                s2f[pl.ds(off, 16)] = jnp.where(
                    valid, -K1 * rc * invr * invr * sg * (1.0 - sg), 0.0)
                uxf[pl.ds(off, 16)] = dx * invr
                uyf[pl.ds(off, 16)] = dy * invr
                uzf[pl.ds(off, 16)] = dz * invr
            cni = jnp.sum(acc)
            plsc.store_scatter(stage, [jnp.full((16,), a, i32)],
                               jnp.full((16,), cni, f32), mask=m0)

        def group(g, _):
            plsc.parallel_loop(g * GROUP, (g + 1) * GROUP)(atom)
            dst = (base + g * GROUP) * NBRS
            pltpu.sync_copy(invrf, invr_h.at[pl.ds(dst, GP)])
            pltpu.sync_copy(s2f, s2_h.at[pl.ds(dst, GP)])
            pltpu.sync_copy(uxf, ux_h.at[pl.ds(dst, GP)])
            pltpu.sync_copy(uyf, uy_h.at[pl.ds(dst, GP)])
            pltpu.sync_copy(uzf, uz_h.at[pl.ds(dst, GP)])
            return 0

        lax.fori_loop(0, NG, group, 0)
        pltpu.sync_copy(stage, cn_out.at[pl.ds(base, APW)])

    pair_sds = jax.ShapeDtypeStruct((N_PAD * NBRS,), f32)
    k1 = pl.kernel(
        k1_body,
        out_type=(jax.ShapeDtypeStruct((N_PAD,), f32),
                  pair_sds, pair_sds, pair_sds, pair_sds, pair_sds),
        mesh=mesh,
        scratch_types=[
            pltpu.VMEM((N_PAD,), f32), pltpu.VMEM((N_PAD,), f32),
            pltpu.VMEM((N_PAD,), f32), pltpu.VMEM((N_PAD,), i32),
            pltpu.VMEM((APW * NBRS,), i32), pltpu.VMEM((NZP,), f32),
            pltpu.VMEM((APW,), f32),
            pltpu.VMEM((GP,), f32), pltpu.VMEM((GP,), f32),
            pltpu.VMEM((GP,), f32), pltpu.VMEM((GP,), f32),
            pltpu.VMEM((GP,), f32),
        ],
        compiler_params=cparams,
    )
    cn_flat, invr_p, s2_p, ux_p, uy_p, uz_p = k1(posx, posy, posz, num,
                                                 nbrf, cv)

    def k2_body(num_h, nbr_h, r4_h, cnr_h, c6r_h, cn_h, invr_h,
                gpart, epart, s1_h,
                nm, nb, r4v, cnv, cnrv, c6buf, gpriv, s1f, invrf, ebuf, sem):
        wid = wid_of()
        base = wid * APW
        pltpu.sync_copy(num_h, nm)
        pltpu.sync_copy(r4_h, r4v)
        pltpu.sync_copy(cnr_h, cnrv)
        pltpu.sync_copy(cn_h, cnv)
        pltpu.sync_copy(nbr_h.at[pl.ds(base * NBRS, APW * NBRS)], nb)
        lane = lax.iota(i32, 16)
        m0 = lane == 0
        lm5 = lane < NRF
        a96 = jnp.where(lm5, lane, NRF - 1) * NZR

        def zero(r, _):
            gpriv[pl.ds(r * 16, 16)] = jnp.zeros((16,), f32)
            return 0
        lax.fori_loop(0, N_PAD // 16, zero, 0)

        def slab(slot):
            return c6buf.at[pl.ds(slot * C6W, C6W)]

        CH = 8
        NCHT = APW // CH

        def issue(q):
            for t in range(CH):
                zi0 = _splat(nm, base + q * CH + t)[0]
                pltpu.make_async_copy(
                    c6r_h.at[zi0], slab(lax.rem(q * CH + t, 2 * CH)),
                    sem).start()

        issue(0)

        def atom(a, eacc):
            i = base + a
            slot = lax.rem(a, 2 * CH)
            t_in_g = lax.rem(a, GROUP)
            zi_v = _splat(nm, i)
            cni = _splat(cnv, i)
            r4i = plsc.load_gather(r4v, [zi_v])
            cnref_i = plsc.load_gather(cnrv, [zi_v + a96])
            dci = cni - cnref_i
            vLi = jnp.exp(K3 * dci * dci)
            vdLi = (2.0 * K3) * dci * vLi
            wi = jnp.sum(jnp.where(lm5, vLi, 0.0))
            dwi = jnp.sum(jnp.where(lm5, vdLi, 0.0))
            Li = [vLi[t] for t in range(NRF)]
            dLi = [vdLi[t] for t in range(NRF)]
            gacc = jnp.zeros((16,), f32)
            for k in range(KV):
                j = nb[pl.ds(a * NBRS + k * 16, 16)]
                valid = (j < N) & (j != i)
                jc = jnp.where(valid, j, 0)
                znj = plsc.load_gather(nm, [jc])
                cnj = plsc.load_gather(cnv, [jc])
                r4j = plsc.load_gather(r4v, [znj])
                invr = invrf[pl.ds(t_in_g * NBRS + k * 16, 16)]
                Lj = []
                dLj = []
                wj = jnp.zeros((16,), f32)
                dwj = jnp.zeros((16,), f32)
                for b in range(NRF):
                    cnref_j = plsc.load_gather(
                        cnrv.at[pl.ds(b * NZR, NZR)], [znj])
                    dcj = cnj - cnref_j
                    ljb = jnp.exp(K3 * dcj * dcj)
                    dljb = (2.0 * K3) * dcj * ljb
                    wj = wj + ljb
                    dwj = dwj + dljb
                    Lj.append(ljb)
                    dLj.append(dljb)
                cslab = slot * C6W
                nsum = jnp.zeros((16,), f32)
                dni = jnp.zeros((16,), f32)
                dnj = jnp.zeros((16,), f32)
                for t in range(NRF):
                    inner = jnp.zeros((16,), f32)
                    innerd = jnp.zeros((16,), f32)
                    for b in range(NRF):
                        ab = t * NRF + b
                        C = plsc.load_gather(
                            c6buf.at[pl.ds(cslab + ab * NZR, NZR)], [znj])
                        inner = inner + C * Lj[b]
                        innerd = innerd + C * dLj[b]
                    nsum = nsum + Li[t] * inner
                    dni = dni + dLi[t] * inner
                    dnj = dnj + Li[t] * innerd
                W = wi * wj
                Wc = jnp.maximum(W, 1e-30)
                live = W > 1e-30
                qq = 3.0 * r4i * r4j
                r0 = qq * _rsqrt(qq)
                fd = A1 * r0 + A2
                f2 = fd * fd
                f6 = f2 * f2 * f2
                f8 = f6 * f2
                i2 = invr * invr
                i6 = i2 * i2 * i2
                i8 = i6 * i2
                dA = 1.0 + f6 * i6
                dB = 1.0 + f8 * i8
                iP6 = 1.0 / (dA * Wc)
                invWc = dA * iP6
                u6 = Wc * iP6
                u8 = 1.0 / dB
                c6 = nsum * invWc
                dc6i = jnp.where(live, (dni - c6 * dwi * wj) * invWc, 0.0)
                dc6j = jnp.where(live, (dnj - c6 * wi * dwj) * invWc, 0.0)
                e6 = c6 * i6 * u6
                e8 = qq * c6 * i8 * u8
                eacc = eacc - jnp.where(valid, S6 * e6 + S8 * e8, 0.0)
                g6 = jnp.where(valid,
                               -0.5 * (S6 * i6 * u6 + S8 * qq * i8 * u8), 0.0)
                gacc = gacc + g6 * dc6i
                plsc.addupdate_scatter(gpriv, [jc], g6 * dc6j)
                s1v = jnp.where(
                    valid,
                    0.5 * (6.0 * S6 * e6 * u6 + 8.0 * S8 * e8 * u8) * invr,
                    0.0)
                s1f[pl.ds(t_in_g * NBRS + k * 16, 16)] = s1v
            gi = jnp.sum(gacc)
            plsc.addupdate_scatter(gpriv, [jnp.full((16,), i, i32)],
                                   jnp.full((16,), gi, f32), mask=m0)
            return eacc

        def chunk(q, eacc):
            for _ in range(CH):
                pltpu.make_async_copy(c6r_h.at[0], slab(0), sem).wait()
            issue(jnp.minimum(q + 1, NCHT - 1))
            eacc = plsc.parallel_loop(q * CH, (q + 1) * CH,
                                      carry=eacc)(atom)
            return eacc

        def group(g, eacc):
            src = (base + g * GROUP) * NBRS
            pltpu.sync_copy(invr_h.at[pl.ds(src, GP)], invrf)
            eacc = lax.fori_loop(g * (GROUP // CH), (g + 1) * (GROUP // CH),
                                 chunk, eacc)
            pltpu.sync_copy(s1f, s1_h.at[pl.ds(src, GP)])
            return eacc

        eacc = lax.fori_loop(0, NG, group, jnp.zeros((16,), f32))
        for _ in range(CH):
            pltpu.make_async_copy(c6r_h.at[0], slab(0), sem).wait()
        ebuf[...] = eacc
        pltpu.sync_copy(ebuf, epart.at[pl.ds(wid * 16, 16)])
        pltpu.sync_copy(gpriv, gpart.at[pl.ds(wid * N_PAD, N_PAD)])

    k2 = pl.kernel(
        k2_body,
        out_type=(
            jax.ShapeDtypeStruct((NW * N_PAD,), f32),
            jax.ShapeDtypeStruct((NW * 16,), f32),
            pair_sds,
        ),
        mesh=mesh,
        scratch_types=[
            pltpu.VMEM((N_PAD,), i32), pltpu.VMEM((APW * NBRS,), i32),
            pltpu.VMEM((NZP,), f32), pltpu.VMEM((N_PAD,), f32),
            pltpu.VMEM((CNW,), f32), pltpu.VMEM((16 * C6W,), f32),
            pltpu.VMEM((N_PAD,), f32),
            pltpu.VMEM((GP,), f32), pltpu.VMEM((GP,), f32),
            pltpu.VMEM((16,), f32),
            pltpu.SemaphoreType.DMA,
        ],
        compiler_params=cparams,
    )
    gpart, epart, s1_p = k2(num, nbrf, r4, cnr, c6r, cn_flat, invr_p)

    def k3_body(nbr_h, s1_h, s2_h, ux_h, uy_h, uz_h, gpart_h,
                fpart,
                nb, gall, gown, gx, gy, gz, s1f, s2f, uxf, uyf, uzf, sem):
        wid = wid_of()
        base = wid * APW
        pltpu.sync_copy(nbr_h.at[pl.ds(base * NBRS, APW * NBRS)], nb)
        lane = lax.iota(i32, 16)
        m0 = lane == 0
        for s in range(NW):
            pltpu.make_async_copy(
                gpart_h.at[pl.ds(s * N_PAD + base, APW)],
                gall.at[pl.ds(s * APW, APW)], sem).start()
        for s in range(NW):
            pltpu.make_async_copy(
                gpart_h.at[pl.ds(base, APW)],
                gall.at[pl.ds(s * APW, APW)], sem).wait()

        def red(r, _):
            acc = jnp.zeros((16,), f32)
            for s in range(NW):
                acc = acc + gall[pl.ds(s * APW + r * 16, 16)]
            gown[pl.ds(r * 16, 16)] = acc
            return 0
        lax.fori_loop(0, APW // 16, red, 0)

        def zero(r, _):
            z = jnp.zeros((16,), f32)
            gx[pl.ds(r * 16, 16)] = z
            gy[pl.ds(r * 16, 16)] = z
            gz[pl.ds(r * 16, 16)] = z
            return 0
        lax.fori_loop(0, N_PAD // 16, zero, 0)

        def atom(a):
            i = base + a
            t_in_g = lax.rem(a, GROUP)
            Gi = _splat(gown, a)
            accx = jnp.zeros((16,), f32)
            accy = jnp.zeros((16,), f32)
            accz = jnp.zeros((16,), f32)
            for k in range(KV):
                off = t_in_g * NBRS + k * 16
                j = nb[pl.ds(a * NBRS + k * 16, 16)]
                F = s1f[pl.ds(off, 16)] + Gi * s2f[pl.ds(off, 16)]
                cx = F * uxf[pl.ds(off, 16)]
                cy = F * uyf[pl.ds(off, 16)]
                cz = F * uzf[pl.ds(off, 16)]
                accx = accx + cx
                accy = accy + cy
                accz = accz + cz
                plsc.addupdate_scatter(gx, [j], cx)
                plsc.addupdate_scatter(gy, [j], cy)
                plsc.addupdate_scatter(gz, [j], cz)
            iv = jnp.full((16,), i, i32)
            plsc.addupdate_scatter(gx, [iv],
                                   jnp.full((16,), -jnp.sum(accx), f32),
                                   mask=m0)
            plsc.addupdate_scatter(gy, [iv],
                                   jnp.full((16,), -jnp.sum(accy), f32),
                                   mask=m0)
            plsc.addupdate_scatter(gz, [iv],
                                   jnp.full((16,), -jnp.sum(accz), f32),
                                   mask=m0)

        def group(g, _):
            src = (base + g * GROUP) * NBRS
            pltpu.sync_copy(s1_h.at[pl.ds(src, GP)], s1f)
            pltpu.sync_copy(s2_h.at[pl.ds(src, GP)], s2f)
            pltpu.sync_copy(ux_h.at[pl.ds(src, GP)], uxf)
            pltpu.sync_copy(uy_h.at[pl.ds(src, GP)], uyf)
            pltpu.sync_copy(uz_h.at[pl.ds(src, GP)], uzf)
            plsc.parallel_loop(g * GROUP, (g + 1) * GROUP)(atom)
            return 0

        lax.fori_loop(0, NG, group, 0)
        pltpu.sync_copy(gx, fpart.at[pl.ds(wid * 3 * N_PAD, N_PAD)])
        pltpu.sync_copy(gy, fpart.at[pl.ds(wid * 3 * N_PAD + N_PAD, N_PAD)])
        pltpu.sync_copy(gz, fpart.at[pl.ds(wid * 3 * N_PAD + 2 * N_PAD, N_PAD)])

    k3 = pl.kernel(
        k3_body,
        out_type=jax.ShapeDtypeStruct((NW * 3 * N_PAD,), f32),
        mesh=mesh,
        scratch_types=[
            pltpu.VMEM((APW * NBRS,), i32),
            pltpu.VMEM((NW * APW,), f32), pltpu.VMEM((APW,), f32),
            pltpu.VMEM((N_PAD,), f32), pltpu.VMEM((N_PAD,), f32),
            pltpu.VMEM((N_PAD,), f32),
            pltpu.VMEM((GP,), f32), pltpu.VMEM((GP,), f32),
            pltpu.VMEM((GP,), f32), pltpu.VMEM((GP,), f32),
            pltpu.VMEM((GP,), f32),
            pltpu.SemaphoreType.DMA,
        ],
        compiler_params=cparams,
    )
    fpart = k3(nbrf, s1_p, s2_p, ux_p, uy_p, uz_p, gpart)

    def k4_body(fpart_h, epart_h, f3, e16, fall, facc, evbuf, ebuf, sem):
        wid = wid_of()
        base = wid * APW
        for c in range(3):
            for s in range(NW):
                pltpu.make_async_copy(
                    fpart_h.at[pl.ds(s * 3 * N_PAD + c * N_PAD + base, APW)],
                    fall.at[pl.ds(s * APW, APW)], sem).start()
            for s in range(NW):
                pltpu.make_async_copy(
                    fpart_h.at[pl.ds(base, APW)],
                    fall.at[pl.ds(s * APW, APW)], sem).wait()

            def red(r, _):
                acc = jnp.zeros((16,), f32)
                for s in range(NW):
                    acc = acc + fall[pl.ds(s * APW + r * 16, 16)]
                facc[pl.ds(r * 16, 16)] = acc * FSCALE
                return 0
            lax.fori_loop(0, APW // 16, red, 0)
            pltpu.sync_copy(facc, f3.at[pl.ds(c * N_PAD + base, APW)])

        @pl.when(wid == 0)
        def _():
            pltpu.sync_copy(epart_h, evbuf)
            acc = jnp.zeros((16,), f32)
            for s in range(NW):
                acc = acc + evbuf[pl.ds(s * 16, 16)]
            e = 0.5 * HARTREE_TO_EV * jnp.sum(acc)
            ebuf[...] = jnp.full((16,), e, f32)
            pltpu.sync_copy(ebuf, e16)

    k4 = pl.kernel(
        k4_body,
        out_type=(
            jax.ShapeDtypeStruct((3 * N_PAD,), f32),
            jax.ShapeDtypeStruct((16,), f32),
        ),
        mesh=mesh,
        scratch_types=[
            pltpu.VMEM((NW * APW,), f32), pltpu.VMEM((APW,), f32),
            pltpu.VMEM((NW * 16,), f32), pltpu.VMEM((16,), f32),
            pltpu.SemaphoreType.DMA,
        ],
        compiler_params=cparams,
    )
    f3, e16 = k4(fpart, epart)

    forces = jnp.transpose(f3.reshape(3, N_PAD))[:N]
    energy = e16[:1]
    cn = cn_flat[:N]
    return (energy, forces, cn)


# speedup vs baseline: 1.3535x; 1.2075x over previous
import jax
import jax.numpy as jnp
from jax import lax
from jax.experimental import pallas as pl
from jax.experimental.pallas import tpu as pltpu
from jax.experimental.pallas import tpu_sc as plsc

BOHR_TO_ANGSTROM = 0.529177210544
ANGSTROM_TO_BOHR = 1.0 / BOHR_TO_ANGSTROM
HARTREE_TO_EV = 27.211386245981
A1 = 0.3981
A2 = 4.4211
S8 = 1.9889
S6 = 1.0
K1 = 16.0
K3 = -4.0


def _rsqrt(x):
    i = plsc.bitcast(x, jnp.int32)
    i = jnp.int32(0x5F3759DF) - lax.shift_right_arithmetic(i, 1)
    y = plsc.bitcast(i, jnp.float32)
    for _ in range(2):
        y = y * (1.5 - 0.5 * x * y * y)
    return y


def _splat(ref, i):
    return plsc.load_gather(ref, [jnp.full((16,), i, jnp.int32)])


def kernel(positions, numbers, neighbor_matrix, covalent_radii, r4r2,
           c6_reference, coord_num_ref):
    N, NBRS = neighbor_matrix.shape
    NZ = covalent_radii.shape[0]
    NRF = coord_num_ref.shape[1]
    KV = NBRS // 16
    try:
        info = plsc.get_sparse_core_info()
        NC, NS = info.num_cores, info.num_subcores
    except ValueError:
        NC, NS = 2, 16
    NW = NC * NS
    GROUP = 64
    N_PAD = -(-N // (NW * GROUP)) * (NW * GROUP)
    APW = N_PAD // NW
    NG = APW // GROUP
    GP = GROUP * NBRS
    NZR = -(-NZ // 32) * 32
    C6W = -(-(NRF * NRF * NZR) // 128) * 128
    NZP = 128
    CNW = NRF * NZR
    FSCALE = -(HARTREE_TO_EV / BOHR_TO_ANGSTROM)

    mesh = plsc.VectorSubcoreMesh(core_axis_name="c", subcore_axis_name="s",
                                  num_cores=NC, num_subcores=NS)
    cparams = pltpu.CompilerParams(needs_layout_passes=False)
    f32, i32 = jnp.float32, jnp.int32

    posb = positions.astype(f32) * ANGSTROM_TO_BOHR
    pad = N_PAD - N
    posx = jnp.pad(posb[:, 0], (0, pad))
    posy = jnp.pad(posb[:, 1], (0, pad))
    posz = jnp.pad(posb[:, 2], (0, pad))
    num = jnp.pad(numbers.astype(i32), (0, pad))
    nbrf = jnp.pad(neighbor_matrix.astype(i32), ((0, pad), (0, 0)),
                   constant_values=N).reshape(-1)
    cv = jnp.pad(covalent_radii.astype(f32), (0, NZP - NZ))
    r4 = jnp.pad(r4r2.astype(f32), (0, NZP - NZ))
    cnr = jnp.pad(coord_num_ref.astype(f32).T,
                  ((0, 0), (0, NZR - NZ))).reshape(-1)
    c6r = jnp.pad(
        jnp.pad(jnp.transpose(
            c6_reference.astype(f32).reshape(NZ, NZ, NRF * NRF), (0, 2, 1)),
            ((0, 0), (0, 0), (0, NZR - NZ))).reshape(NZ, NRF * NRF * NZR),
        ((0, 0), (0, C6W - NRF * NRF * NZR)))

    def wid_of():
        return lax.axis_index("s") * NC + lax.axis_index("c")

    def k1_body(posx_h, posy_h, posz_h, num_h, nbr_h, cv_h,
                cn_out, invr_h, s2_h, ux_h, uy_h, uz_h,
                px, py, pz, nm, nb, cvv, stage, invrf, s2f, uxf, uyf, uzf):
        wid = wid_of()
        base = wid * APW
        pltpu.sync_copy(posx_h, px)
        pltpu.sync_copy(posy_h, py)
        pltpu.sync_copy(posz_h, pz)
        pltpu.sync_copy(num_h, nm)
        pltpu.sync_copy(cv_h, cvv)
        pltpu.sync_copy(nbr_h.at[pl.ds(base * NBRS, APW * NBRS)], nb)
        lane = lax.iota(i32, 16)
        m0 = lane == 0

        def atom(a):
            i = base + a
            t = lax.rem(a, GROUP)
            xi = _splat(px, i)
            yi = _splat(py, i)
            zi = _splat(pz, i)
            zn = _splat(nm, i)
            rci = plsc.load_gather(cvv, [zn])
            acc = jnp.zeros((16,), f32)
            for k in range(KV):
                j = nb[pl.ds(a * NBRS + k * 16, 16)]
                valid = (j < N) & (j != i)
                jc = jnp.where(valid, j, 0)
                xj = plsc.load_gather(px, [jc])
                yj = plsc.load_gather(py, [jc])
                zj = plsc.load_gather(pz, [jc])
                znj = plsc.load_gather(nm, [jc])
                rcj = plsc.load_gather(cvv, [znj])
                dx = xj - xi
                dy = yj - yi
                dz = zj - zi
                d2 = dx * dx + dy * dy + dz * dz
                d2s = jnp.where(valid, d2, 1.0)
                invr = _rsqrt(d2s)
                rc = rci + rcj
                ex = jnp.exp(-K1 * (rc * invr - 1.0))
                sg = 1.0 / (1.0 + ex)
                acc = acc + jnp.where(valid, sg, 0.0)
                off = t * NBRS + k * 16
                invrf[pl.ds(off, 16)] = invr
                s2f[pl.ds(off, 16)] = jnp.where(
                    valid, -K1 * rc * invr * invr * sg * (1.0 - sg), 0.0)
                uxf[pl.ds(off, 16)] = dx * invr
                uyf[pl.ds(off, 16)] = dy * invr
                uzf[pl.ds(off, 16)] = dz * invr
            cni = jnp.sum(acc)
            plsc.store_scatter(stage, [jnp.full((16,), a, i32)],
                               jnp.full((16,), cni, f32), mask=m0)

        def group(g, _):
            plsc.parallel_loop(g * GROUP, (g + 1) * GROUP)(atom)
            dst = (base + g * GROUP) * NBRS
            pltpu.sync_copy(invrf, invr_h.at[pl.ds(dst, GP)])
            pltpu.sync_copy(s2f, s2_h.at[pl.ds(dst, GP)])
            pltpu.sync_copy(uxf, ux_h.at[pl.ds(dst, GP)])
            pltpu.sync_copy(uyf, uy_h.at[pl.ds(dst, GP)])
            pltpu.sync_copy(uzf, uz_h.at[pl.ds(dst, GP)])
            return 0

        lax.fori_loop(0, NG, group, 0)
        pltpu.sync_copy(stage, cn_out.at[pl.ds(base, APW)])

    pair_sds = jax.ShapeDtypeStruct((N_PAD * NBRS,), f32)
    k1 = pl.kernel(
        k1_body,
        out_type=(jax.ShapeDtypeStruct((N_PAD,), f32),
                  pair_sds, pair_sds, pair_sds, pair_sds, pair_sds),
        mesh=mesh,
        scratch_types=[
            pltpu.VMEM((N_PAD,), f32), pltpu.VMEM((N_PAD,), f32),
            pltpu.VMEM((N_PAD,), f32), pltpu.VMEM((N_PAD,), i32),
            pltpu.VMEM((APW * NBRS,), i32), pltpu.VMEM((NZP,), f32),
            pltpu.VMEM((APW,), f32),
            pltpu.VMEM((GP,), f32), pltpu.VMEM((GP,), f32),
            pltpu.VMEM((GP,), f32), pltpu.VMEM((GP,), f32),
            pltpu.VMEM((GP,), f32),
        ],
        compiler_params=cparams,
    )
    cn_flat, invr_p, s2_p, ux_p, uy_p, uz_p = k1(posx, posy, posz, num,
                                                 nbrf, cv)

    def k2_body(num_h, nbr_h, r4_h, cnr_h, c6r_h, cn_h, invr_h,
                gpart, epart, s1_h,
                nm, nb, r4v, cnv, cnrv, c6buf, gpriv, s1f, invrf, ebuf, sem):
        wid = wid_of()
        base = wid * APW
        pltpu.sync_copy(num_h, nm)
        pltpu.sync_copy(r4_h, r4v)
        pltpu.sync_copy(cnr_h, cnrv)
        pltpu.sync_copy(cn_h, cnv)
        pltpu.sync_copy(nbr_h.at[pl.ds(base * NBRS, APW * NBRS)], nb)
        lane = lax.iota(i32, 16)
        m0 = lane == 0
        lm5 = lane < NRF
        a96 = jnp.where(lm5, lane, NRF - 1) * NZR

        def zero(r, _):
            gpriv[pl.ds(r * 16, 16)] = jnp.zeros((16,), f32)
            return 0
        lax.fori_loop(0, N_PAD // 16, zero, 0)

        def slab(slot):
            return c6buf.at[pl.ds(slot * C6W, C6W)]

        for t in range(4):
            zi0 = _splat(nm, base + t)[0]
            pltpu.make_async_copy(c6r_h.at[zi0], slab(t), sem).start()

        def atom(a, eacc):
            i = base + a
            slot = lax.rem(a, 8)
            t_in_g = lax.rem(a, GROUP)
            pltpu.make_async_copy(c6r_h.at[0], slab(slot), sem).wait()
            zi_v = _splat(nm, i)
            cni = _splat(cnv, i)
            r4i = plsc.load_gather(r4v, [zi_v])
            cnref_i = plsc.load_gather(cnrv, [zi_v + a96])
            dci = cni - cnref_i
            vLi = jnp.exp(K3 * dci * dci)
            vdLi = (2.0 * K3) * dci * vLi
            wi = jnp.sum(jnp.where(lm5, vLi, 0.0))
            dwi = jnp.sum(jnp.where(lm5, vdLi, 0.0))
            Li = [vLi[t] for t in range(NRF)]
            dLi = [vdLi[t] for t in range(NRF)]
            gacc = jnp.zeros((16,), f32)
            for k in range(KV):
                j = nb[pl.ds(a * NBRS + k * 16, 16)]
                valid = (j < N) & (j != i)
                jc = jnp.where(valid, j, 0)
                znj = plsc.load_gather(nm, [jc])
                cnj = plsc.load_gather(cnv, [jc])
                r4j = plsc.load_gather(r4v, [znj])
                invr = invrf[pl.ds(t_in_g * NBRS + k * 16, 16)]
                Lj = []
                dLj = []
                wj = jnp.zeros((16,), f32)
                dwj = jnp.zeros((16,), f32)
                for b in range(NRF):
                    cnref_j = plsc.load_gather(
                        cnrv.at[pl.ds(b * NZR, NZR)], [znj])
                    dcj = cnj - cnref_j
                    ljb = jnp.exp(K3 * dcj * dcj)
                    dljb = (2.0 * K3) * dcj * ljb
                    wj = wj + ljb
                    dwj = dwj + dljb
                    Lj.append(ljb)
                    dLj.append(dljb)
                cslab = slot * C6W
                nsum = jnp.zeros((16,), f32)
                dni = jnp.zeros((16,), f32)
                dnj = jnp.zeros((16,), f32)
                for t in range(NRF):
                    inner = jnp.zeros((16,), f32)
                    innerd = jnp.zeros((16,), f32)
                    for b in range(NRF):
                        ab = t * NRF + b
                        C = plsc.load_gather(
                            c6buf.at[pl.ds(cslab + ab * NZR, NZR)], [znj])
                        inner = inner + C * Lj[b]
                        innerd = innerd + C * dLj[b]
                    nsum = nsum + Li[t] * inner
                    dni = dni + dLi[t] * inner
                    dnj = dnj + Li[t] * innerd
                W = wi * wj
                Wc = jnp.maximum(W, 1e-30)
                live = W > 1e-30
                qq = 3.0 * r4i * r4j
                r0 = qq * _rsqrt(qq)
                fd = A1 * r0 + A2
                f2 = fd * fd
                f6 = f2 * f2 * f2
                f8 = f6 * f2
                i2 = invr * invr
                i6 = i2 * i2 * i2
                i8 = i6 * i2
                dA = 1.0 + f6 * i6
                dB = 1.0 + f8 * i8
                iP6 = 1.0 / (dA * Wc)
                invWc = dA * iP6
                u6 = Wc * iP6
                u8 = 1.0 / dB
                c6 = nsum * invWc
                dc6i = jnp.where(live, (dni - c6 * dwi * wj) * invWc, 0.0)
                dc6j = jnp.where(live, (dnj - c6 * wi * dwj) * invWc, 0.0)
                e6 = c6 * i6 * u6
                e8 = qq * c6 * i8 * u8
                eacc = eacc - jnp.where(valid, S6 * e6 + S8 * e8, 0.0)
                g6 = jnp.where(valid,
                               -0.5 * (S6 * i6 * u6 + S8 * qq * i8 * u8), 0.0)
                gacc = gacc + g6 * dc6i
                plsc.addupdate_scatter(gpriv, [jc], g6 * dc6j)
                s1v = jnp.where(
                    valid,
                    0.5 * (6.0 * S6 * e6 * u6 + 8.0 * S8 * e8 * u8) * invr,
                    0.0)
                s1f[pl.ds(t_in_g * NBRS + k * 16, 16)] = s1v
            gi = jnp.sum(gacc)
            plsc.addupdate_scatter(gpriv, [jnp.full((16,), i, i32)],
                                   jnp.full((16,), gi, f32), mask=m0)
            nx = jnp.minimum(a + 4, APW - 1)
            zin = _splat(nm, base + nx)[0]
            pltpu.make_async_copy(c6r_h.at[zin], slab(lax.rem(a + 4, 8)),
                                  sem).start()
            return eacc

        def group(g, eacc):
            src = (base + g * GROUP) * NBRS
            pltpu.sync_copy(invr_h.at[pl.ds(src, GP)], invrf)
            eacc = lax.fori_loop(g * GROUP, (g + 1) * GROUP, atom, eacc)
            pltpu.sync_copy(s1f, s1_h.at[pl.ds(src, GP)])
            return eacc

        eacc = lax.fori_loop(0, NG, group, jnp.zeros((16,), f32))
        for _ in range(4):
            pltpu.make_async_copy(c6r_h.at[0], slab(0), sem).wait()
        ebuf[...] = eacc
        pltpu.sync_copy(ebuf, epart.at[pl.ds(wid * 16, 16)])
        pltpu.sync_copy(gpriv, gpart.at[pl.ds(wid * N_PAD, N_PAD)])

    k2 = pl.kernel(
        k2_body,
        out_type=(
            jax.ShapeDtypeStruct((NW * N_PAD,), f32),
            jax.ShapeDtypeStruct((NW * 16,), f32),
            pair_sds,
        ),
        mesh=mesh,
        scratch_types=[
            pltpu.VMEM((N_PAD,), i32), pltpu.VMEM((APW * NBRS,), i32),
            pltpu.VMEM((NZP,), f32), pltpu.VMEM((N_PAD,), f32),
            pltpu.VMEM((CNW,), f32), pltpu.VMEM((8 * C6W,), f32),
            pltpu.VMEM((N_PAD,), f32),
            pltpu.VMEM((GP,), f32), pltpu.VMEM((GP,), f32),
            pltpu.VMEM((16,), f32),
            pltpu.SemaphoreType.DMA,
        ],
        compiler_params=cparams,
    )
    gpart, epart, s1_p = k2(num, nbrf, r4, cnr, c6r, cn_flat, invr_p)

    def k3_body(nbr_h, s1_h, s2_h, ux_h, uy_h, uz_h, gpart_h,
                fpart,
                nb, gall, gown, gx, gy, gz, s1f, s2f, uxf, uyf, uzf, sem):
        wid = wid_of()
        base = wid * APW
        pltpu.sync_copy(nbr_h.at[pl.ds(base * NBRS, APW * NBRS)], nb)
        lane = lax.iota(i32, 16)
        m0 = lane == 0
        for s in range(NW):
            pltpu.make_async_copy(
                gpart_h.at[pl.ds(s * N_PAD + base, APW)],
                gall.at[pl.ds(s * APW, APW)], sem).start()
        for s in range(NW):
            pltpu.make_async_copy(
                gpart_h.at[pl.ds(base, APW)],
                gall.at[pl.ds(s * APW, APW)], sem).wait()

        def red(r, _):
            acc = jnp.zeros((16,), f32)
            for s in range(NW):
                acc = acc + gall[pl.ds(s * APW + r * 16, 16)]
            gown[pl.ds(r * 16, 16)] = acc
            return 0
        lax.fori_loop(0, APW // 16, red, 0)

        def zero(r, _):
            z = jnp.zeros((16,), f32)
            gx[pl.ds(r * 16, 16)] = z
            gy[pl.ds(r * 16, 16)] = z
            gz[pl.ds(r * 16, 16)] = z
            return 0
        lax.fori_loop(0, N_PAD // 16, zero, 0)

        def atom(a):
            i = base + a
            t_in_g = lax.rem(a, GROUP)
            Gi = _splat(gown, a)
            accx = jnp.zeros((16,), f32)
            accy = jnp.zeros((16,), f32)
            accz = jnp.zeros((16,), f32)
            for k in range(KV):
                off = t_in_g * NBRS + k * 16
                j = nb[pl.ds(a * NBRS + k * 16, 16)]
                F = s1f[pl.ds(off, 16)] + Gi * s2f[pl.ds(off, 16)]
                cx = F * uxf[pl.ds(off, 16)]
                cy = F * uyf[pl.ds(off, 16)]
                cz = F * uzf[pl.ds(off, 16)]
                accx = accx + cx
                accy = accy + cy
                accz = accz + cz
                plsc.addupdate_scatter(gx, [j], cx)
                plsc.addupdate_scatter(gy, [j], cy)
                plsc.addupdate_scatter(gz, [j], cz)
            iv = jnp.full((16,), i, i32)
            plsc.addupdate_scatter(gx, [iv],
                                   jnp.full((16,), -jnp.sum(accx), f32),
                                   mask=m0)
            plsc.addupdate_scatter(gy, [iv],
                                   jnp.full((16,), -jnp.sum(accy), f32),
                                   mask=m0)
            plsc.addupdate_scatter(gz, [iv],
                                   jnp.full((16,), -jnp.sum(accz), f32),
                                   mask=m0)

        def group(g, _):
            src = (base + g * GROUP) * NBRS
            pltpu.sync_copy(s1_h.at[pl.ds(src, GP)], s1f)
            pltpu.sync_copy(s2_h.at[pl.ds(src, GP)], s2f)
            pltpu.sync_copy(ux_h.at[pl.ds(src, GP)], uxf)
            pltpu.sync_copy(uy_h.at[pl.ds(src, GP)], uyf)
            pltpu.sync_copy(uz_h.at[pl.ds(src, GP)], uzf)
            plsc.parallel_loop(g * GROUP, (g + 1) * GROUP)(atom)
            return 0

        lax.fori_loop(0, NG, group, 0)
        pltpu.sync_copy(gx, fpart.at[pl.ds(wid * 3 * N_PAD, N_PAD)])
        pltpu.sync_copy(gy, fpart.at[pl.ds(wid * 3 * N_PAD + N_PAD, N_PAD)])
        pltpu.sync_copy(gz, fpart.at[pl.ds(wid * 3 * N_PAD + 2 * N_PAD, N_PAD)])

    k3 = pl.kernel(
        k3_body,
        out_type=jax.ShapeDtypeStruct((NW * 3 * N_PAD,), f32),
        mesh=mesh,
        scratch_types=[
            pltpu.VMEM((APW * NBRS,), i32),
            pltpu.VMEM((NW * APW,), f32), pltpu.VMEM((APW,), f32),
            pltpu.VMEM((N_PAD,), f32), pltpu.VMEM((N_PAD,), f32),
            pltpu.VMEM((N_PAD,), f32),
            pltpu.VMEM((GP,), f32), pltpu.VMEM((GP,), f32),
            pltpu.VMEM((GP,), f32), pltpu.VMEM((GP,), f32),
            pltpu.VMEM((GP,), f32),
            pltpu.SemaphoreType.DMA,
        ],
        compiler_params=cparams,
    )
    fpart = k3(nbrf, s1_p, s2_p, ux_p, uy_p, uz_p, gpart)

    def k4_body(fpart_h, epart_h, f3, e16, fall, facc, evbuf, ebuf, sem):
        wid = wid_of()
        base = wid * APW
        for c in range(3):
            for s in range(NW):
                pltpu.make_async_copy(
                    fpart_h.at[pl.ds(s * 3 * N_PAD + c * N_PAD + base, APW)],
                    fall.at[pl.ds(s * APW, APW)], sem).start()
            for s in range(NW):
                pltpu.make_async_copy(
                    fpart_h.at[pl.ds(base, APW)],
                    fall.at[pl.ds(s * APW, APW)], sem).wait()

            def red(r, _):
                acc = jnp.zeros((16,), f32)
                for s in range(NW):
                    acc = acc + fall[pl.ds(s * APW + r * 16, 16)]
                facc[pl.ds(r * 16, 16)] = acc * FSCALE
                return 0
            lax.fori_loop(0, APW // 16, red, 0)
            pltpu.sync_copy(facc, f3.at[pl.ds(c * N_PAD + base, APW)])

        @pl.when(wid == 0)
        def _():
            pltpu.sync_copy(epart_h, evbuf)
            acc = jnp.zeros((16,), f32)
            for s in range(NW):
                acc = acc + evbuf[pl.ds(s * 16, 16)]
            e = 0.5 * HARTREE_TO_EV * jnp.sum(acc)
            ebuf[...] = jnp.full((16,), e, f32)
            pltpu.sync_copy(ebuf, e16)

    k4 = pl.kernel(
        k4_body,
        out_type=(
            jax.ShapeDtypeStruct((3 * N_PAD,), f32),
            jax.ShapeDtypeStruct((16,), f32),
        ),
        mesh=mesh,
        scratch_types=[
            pltpu.VMEM((NW * APW,), f32), pltpu.VMEM((APW,), f32),
            pltpu.VMEM((NW * 16,), f32), pltpu.VMEM((16,), f32),
            pltpu.SemaphoreType.DMA,
        ],
        compiler_params=cparams,
    )
    f3, e16 = k4(fpart, epart)

    forces = jnp.transpose(f3.reshape(3, N_PAD))[:N]
    energy = e16[:1]
    cn = cn_flat[:N]
    return (energy, forces, cn)
